# baseline (speedup 1.0000x reference)
import jax
import jax.numpy as jnp
from jax import lax
from jax.experimental import pallas as pl
from jax.experimental.pallas import tpu as pltpu
from jax.experimental.pallas import tpu_sc as plsc

_BATCH = 16384
_F = 64
_NC = 2
_NS = 16
_NW = _NC * _NS
_BPW = _BATCH // _NW
_CHUNK = 128
_NCHUNK = _BPW // _CHUNK


def _mf_body(uid_hbm, iid_hbm, p_hbm, q_hbm, bu_hbm, bi_hbm, out_hbm,
             uidx, iidx, uhi, ihi, prows, qrows, burows, birows, outv, sem):
    wid = lax.axis_index("s") * _NC + lax.axis_index("c")
    base = wid * _BPW

    for j in range(_NCHUNK):
        pltpu.sync_copy(uid_hbm.at[pl.ds(base + j * _CHUNK, _CHUNK)], uidx.at[j])
        pltpu.sync_copy(iid_hbm.at[pl.ds(base + j * _CHUNK, _CHUNK)], iidx.at[j])

    for j in range(_NCHUNK):
        for t in range(_CHUNK // 16):
            sl = pl.ds(t * 16, 16)
            uhi.at[j][sl] = jax.lax.shift_right_logical(uidx.at[j][sl], 4)
            ihi.at[j][sl] = jax.lax.shift_right_logical(iidx.at[j][sl], 4)

    copies = []
    for j in range(_NCHUNK):
        sl = pl.ds(j * _CHUNK, _CHUNK)
        copies.append(pltpu.async_copy(p_hbm.at[uidx.at[j]], prows.at[sl], sem))
        copies.append(pltpu.async_copy(q_hbm.at[iidx.at[j]], qrows.at[sl], sem))
        copies.append(pltpu.async_copy(bu_hbm.at[uhi.at[j]], burows.at[sl], sem))
        copies.append(pltpu.async_copy(bi_hbm.at[ihi.at[j]], birows.at[sl], sem))
    for cp in copies:
        cp.wait()

    lanes = lax.iota(jnp.int32, 16)

    def group(g, carry):
        rb = g * 16
        j = g // (_CHUNK // 16)
        o = (g % (_CHUNK // 16)) * 16
        rows = rb + lanes
        uvals = uidx.at[j][pl.ds(o, 16)]
        ivals = iidx.at[j][pl.ds(o, 16)]
        bu_v = plsc.load_gather(burows, [rows, jnp.bitwise_and(uvals, 15)])
        bi_v = plsc.load_gather(birows, [rows, jnp.bitwise_and(ivals, 15)])
        sums = bu_v + bi_v
        for i in range(16):
            r = rb + i
            a = prows[r, 0:16] * qrows[r, 0:16]
            for k in range(1, _F // 16):
                a = a + prows[r, 16 * k:16 * k + 16] * qrows[r, 16 * k:16 * k + 16]
            sums = jnp.where(lanes == i, jnp.sum(a) + sums, sums)
        outv[pl.ds(rb, 16)] = sums
        return carry

    lax.fori_loop(0, _BPW // 16, group, 0)

    pltpu.sync_copy(outv, out_hbm.at[pl.ds(base, _BPW)])


@jax.jit
def kernel(user_id, item_id, P, Q, user_bias, item_bias):
    mesh = plsc.VectorSubcoreMesh(core_axis_name="c", subcore_axis_name="s")
    run = pl.kernel(
        _mf_body,
        out_type=jax.ShapeDtypeStruct((_BATCH,), jnp.float32),
        mesh=mesh,
        compiler_params=pltpu.CompilerParams(
            needs_layout_passes=False, use_tc_tiling_on_sc=False),
        scratch_types=[
            pltpu.VMEM((_NCHUNK, _CHUNK), jnp.int32),
            pltpu.VMEM((_NCHUNK, _CHUNK), jnp.int32),
            pltpu.VMEM((_NCHUNK, _CHUNK), jnp.int32),
            pltpu.VMEM((_NCHUNK, _CHUNK), jnp.int32),
            pltpu.VMEM((_BPW, _F), jnp.float32),
            pltpu.VMEM((_BPW, _F), jnp.float32),
            pltpu.VMEM((_BPW, 16), jnp.float32),
            pltpu.VMEM((_BPW, 16), jnp.float32),
            pltpu.VMEM((_BPW,), jnp.float32),
            pltpu.SemaphoreType.DMA,
        ],
    )
    return run(user_id, item_id, P, Q,
               user_bias.reshape(-1, 16), item_bias.reshape(-1, 16))
